# Initial kernel scaffold; baseline (speedup 1.0000x reference)
import jax
import jax.numpy as jnp
from jax.experimental import pallas as pl


def kernel(x, edge_index, edge_attr, params):
    raise NotImplementedError("write your pallas kernel here")


# speedup vs baseline: 3.6351x; 3.6351x over previous
import functools

import jax
import jax.numpy as jnp
from jax import lax
from jax.experimental import pallas as pl
from jax.experimental.pallas import tpu as pltpu
from jax.experimental.pallas import tpu_sc as plsc

_N = 10000
_E = 320000
_D = 128
_EPS = 1e-5

_NC = 2
_NS = 16
_LANES = 16
_CHUNK = 128
_NCHUNK = _E // _CHUNK
_CPC = _NCHUNK // _NC
_NP = 10240
_RPS = _NP // _NS


def _zero_rows(buf, rows, cols):
    zv = jnp.zeros((_LANES,), jnp.float32)

    def row(i, _):
        for k in range(cols // _LANES):
            buf[i, pl.ds(k * _LANES, _LANES)] = zv
        return 0

    lax.fori_loop(0, rows, row, 0)


def _mesh():
    return plsc.VectorSubcoreMesh(core_axis_name="c", subcore_axis_name="s",
                                  num_cores=_NC, num_subcores=_NS)


@functools.cache
def _make_edge_pass():
    scratch = [
        pltpu.VMEM((1, _CHUNK), jnp.int32),
        pltpu.VMEM((1, _CHUNK), jnp.int32),
        pltpu.VMEM((_CHUNK, _D), jnp.float32),
        pltpu.VMEM((_CHUNK, _D), jnp.float32),
        pltpu.VMEM_SHARED((_NP, _D), jnp.float32),
        pltpu.SemaphoreType.DMA,
    ]

    def body(src_hbm, dst_hbm, x_hbm, ea_hbm, part_hbm,
             srcb, dstb, xb, eb, acc, sem):
        c = lax.axis_index("c")
        s = lax.axis_index("s")

        _zero_rows(xb, _CHUNK, _D)
        rbase = s * _RPS
        for j in range(_RPS // _CHUNK):
            pltpu.sync_copy(xb, acc.at[pl.ds(rbase + j * _CHUNK, _CHUNK)])
        plsc.subcore_barrier()

        base = c * _CPC + s
        nch = (_CPC - s + _NS - 1) // _NS

        def step(i, _):
            q = base + i * _NS
            pltpu.sync_copy(src_hbm.at[q], srcb)
            pltpu.sync_copy(dst_hbm.at[q], dstb)
            pltpu.async_copy(x_hbm.at[srcb.at[0]], xb, sem).wait()
            pltpu.sync_copy(ea_hbm.at[pl.ds(q * _CHUNK, _CHUNK)], eb)

            def mulrow(j, _):
                for k in range(_D // _LANES):
                    sl = pl.ds(k * _LANES, _LANES)
                    eb[j, sl] = eb[j, sl] * xb[j, sl]
                return 0

            lax.fori_loop(0, _CHUNK, mulrow, 0)
            pltpu.sync_copy(eb, acc.at[dstb.at[0]], add=True)
            return 0

        lax.fori_loop(0, nch, step, 0)
        plsc.subcore_barrier()

        pltpu.sync_copy(acc.at[pl.ds(rbase, _RPS)],
                        part_hbm.at[c, pl.ds(rbase, _RPS)])

    return pl.kernel(
        body,
        out_type=jax.ShapeDtypeStruct((_NC, _NP, _D), jnp.float32),
        mesh=_mesh(),
        scratch_types=scratch,
    )


@functools.cache
def _make_count_pass():
    scratch = [
        pltpu.VMEM((1, _CHUNK), jnp.int32),
        pltpu.VMEM((_CHUNK, _D), jnp.float32),
        pltpu.VMEM_SHARED((_NP, _D), jnp.float32),
    ]

    def body(dst_hbm, cntp_hbm, dstb, oneb, accc):
        c = lax.axis_index("c")
        s = lax.axis_index("s")

        _zero_rows(oneb, _CHUNK, _D)
        rbase = s * _RPS
        for j in range(_RPS // _CHUNK):
            pltpu.sync_copy(oneb, accc.at[pl.ds(rbase + j * _CHUNK, _CHUNK)])
        ov = jnp.ones((_LANES,), jnp.float32)

        def onerow(i, _):
            oneb[i, pl.ds(0, _LANES)] = ov
            return 0

        lax.fori_loop(0, _CHUNK, onerow, 0)
        plsc.subcore_barrier()

        base = c * _CPC + s
        nch = (_CPC - s + _NS - 1) // _NS

        def step(i, _):
            q = base + i * _NS
            pltpu.sync_copy(dst_hbm.at[q], dstb)
            pltpu.sync_copy(oneb, accc.at[dstb.at[0]], add=True)
            return 0

        lax.fori_loop(0, nch, step, 0)
        plsc.subcore_barrier()

        pltpu.sync_copy(accc.at[pl.ds(rbase, _RPS)],
                        cntp_hbm.at[c, pl.ds(rbase, _RPS)])

    return pl.kernel(
        body,
        out_type=jax.ShapeDtypeStruct((_NC, _NP, _D), jnp.float32),
        mesh=_mesh(),
        scratch_types=scratch,
    )


_BE = 2000


def _rel_chain_body(ea_ref, w0, w1, w2, b0, b1, b2, o1, o2, o3):
    e1 = jnp.dot(ea_ref[...], w0[...],
                 preferred_element_type=jnp.float32) + b0[...]
    e2 = jnp.dot(e1, w1[...], preferred_element_type=jnp.float32) + b1[...]
    e3 = jnp.dot(e2, w2[...], preferred_element_type=jnp.float32) + b2[...]
    o1[...] = e1
    o2[...] = e2
    o3[...] = e3


def _rel_chain(ea, w0t, w1t, w2t, b0, b1, b2):
    ew = pl.BlockSpec((_BE, _D), lambda i: (i, 0))
    wf = pl.BlockSpec((_D, _D), lambda i: (0, 0))
    bf = pl.BlockSpec((1, _D), lambda i: (0, 0))
    return pl.pallas_call(
        _rel_chain_body,
        grid=(_E // _BE,),
        in_specs=[ew, wf, wf, wf, bf, bf, bf],
        out_specs=[ew, ew, ew],
        out_shape=[jax.ShapeDtypeStruct((_E, _D), jnp.float32)] * 3,
    )(ea, w0t, w1t, w2t, b0, b1, b2)


def _combine_body(apply_bn, part, cntp, x, wint, wselft, b_in, b_self,
                  *rest):
    if apply_bn:
        bn_g, bn_b, out = rest
    else:
        (out,) = rest
    seg = part[0, :_N] + part[1, :_N]
    cnt = cntp[0, :_N, 0:1] + cntp[1, :_N, 0:1]
    denom = jnp.maximum(cnt, 1.0)
    agg = (jnp.dot(seg, wint[...], preferred_element_type=jnp.float32) / denom
           + jnp.where(cnt > 0, 1.0, 0.0) * b_in[...])
    val = agg + jnp.dot(x[...], wselft[...],
                        preferred_element_type=jnp.float32) + b_self[...]
    if apply_bn:
        mu = jnp.mean(val, axis=0, keepdims=True)
        var = jnp.mean((val - mu) ** 2, axis=0, keepdims=True)
        val = (val - mu) * lax.rsqrt(var + _EPS) * bn_g[...] + bn_b[...]
        val = jnp.maximum(val, 0.0)
    out[...] = val


def _combine(apply_bn, part, cntp, x, wint, wselft, b_in, b_self, *bn):
    return pl.pallas_call(
        functools.partial(_combine_body, apply_bn),
        out_shape=jax.ShapeDtypeStruct((_N, _D), jnp.float32),
    )(part, cntp, x, wint, wselft, b_in, b_self, *bn)


def kernel(x, edge_index, edge_attr, params):
    src = edge_index[0].reshape(_NCHUNK, 1, _CHUNK)
    dst = edge_index[1].reshape(_NCHUNK, 1, _CHUNK)
    p = params
    row = lambda v: v.reshape(1, _D)

    ea1, ea2, ea3 = _rel_chain(
        edge_attr,
        p["W_rel_0"].T, p["W_rel_1"].T, p["W_rel_2"].T,
        row(p["b_rel_0"]), row(p["b_rel_1"]), row(p["b_rel_2"]),
    )

    cntp = _make_count_pass()(dst)
    part0 = _make_edge_pass()(src, dst, x, edge_attr)
    x1 = _combine(True, part0, cntp, x,
                  p["W_in_0"].T, p["W_self_0"].T,
                  row(p["b_in_0"]), row(p["b_self_0"]),
                  row(p["bn_g_0"]), row(p["bn_b_0"]))

    part1 = _make_edge_pass()(src, dst, x1, ea1)
    x2 = _combine(True, part1, cntp, x1,
                  p["W_in_1"].T, p["W_self_1"].T,
                  row(p["b_in_1"]), row(p["b_self_1"]),
                  row(p["bn_g_1"]), row(p["bn_b_1"]))

    part2 = _make_edge_pass()(src, dst, x2, ea2)
    x3 = _combine(False, part2, cntp, x2,
                  p["W_in_2"].T, p["W_self_2"].T,
                  row(p["b_in_2"]), row(p["b_self_2"]))

    return (x3, ea3)


# speedup vs baseline: 5.5314x; 1.5216x over previous
import functools

import jax
import jax.numpy as jnp
from jax import lax
from jax.experimental import pallas as pl
from jax.experimental.pallas import tpu as pltpu
from jax.experimental.pallas import tpu_sc as plsc

_N = 10000
_E = 320000
_D = 128
_EPS = 1e-5

_NC = 2
_NS = 16
_LANES = 16
_CHUNK = 80
_NCHUNK = _E // _CHUNK
_CPC = _NCHUNK // _NC
_CW = _CPC // _NS
_NP = 10240
_RPS = _NP // _NS


def _zero_rows(buf, rows, cols):
    zv = jnp.zeros((_LANES,), jnp.float32)

    def row(i, _):
        for k in range(cols // _LANES):
            buf[i, pl.ds(k * _LANES, _LANES)] = zv
        return 0

    lax.fori_loop(0, rows, row, 0)


def _mesh():
    return plsc.VectorSubcoreMesh(core_axis_name="c", subcore_axis_name="s",
                                  num_cores=_NC, num_subcores=_NS)


@functools.cache
def _make_edge_pass():
    scratch = [
        pltpu.VMEM((1, _CHUNK), jnp.int32),
        pltpu.VMEM((1, _CHUNK), jnp.int32),
        pltpu.VMEM((1, _CHUNK), jnp.int32),
        pltpu.VMEM((1, _CHUNK), jnp.int32),
        pltpu.VMEM((_CHUNK, _D), jnp.float32),
        pltpu.VMEM((_CHUNK, _D), jnp.float32),
        pltpu.VMEM((_CHUNK, _D), jnp.float32),
        pltpu.VMEM((_CHUNK, _D), jnp.float32),
        pltpu.VMEM_SHARED((_NP, _D), jnp.float32),
        pltpu.SemaphoreType.DMA,
        pltpu.SemaphoreType.DMA,
        pltpu.SemaphoreType.DMA,
        pltpu.SemaphoreType.DMA,
        pltpu.SemaphoreType.DMA,
        pltpu.SemaphoreType.DMA,
        pltpu.SemaphoreType.DMA,
        pltpu.SemaphoreType.DMA,
    ]

    def body(src_hbm, dst_hbm, x_hbm, ea_hbm, part_hbm,
             sb0, sb1, db0, db1, xb0, xb1, eb0, eb1, acc,
             ss0, ss1, sd0, sd1, sx0, sx1, se0, se1):
        c = lax.axis_index("c")
        s = lax.axis_index("s")
        sb = (sb0, sb1)
        db = (db0, db1)
        xb = (xb0, xb1)
        eb = (eb0, eb1)
        ss = (ss0, ss1)
        sd = (sd0, sd1)
        sx = (sx0, sx1)
        se = (se0, se1)

        _zero_rows(xb0, _CHUNK, _D)
        rbase = s * _RPS
        for j in range(_RPS // _CHUNK):
            pltpu.sync_copy(xb0, acc.at[pl.ds(rbase + j * _CHUNK, _CHUNK)])
        plsc.subcore_barrier()

        start = c * _CPC + s * _CW

        def issue_idx(i, b):
            pltpu.async_copy(src_hbm.at[start + i], sb[b], ss[b])
            pltpu.async_copy(dst_hbm.at[start + i], db[b], sd[b])

        def wait_idx(i, b):
            pltpu.make_async_copy(src_hbm.at[start + i], sb[b], ss[b]).wait()
            pltpu.make_async_copy(dst_hbm.at[start + i], db[b], sd[b]).wait()

        def issue_data(i, b):
            pltpu.async_copy(x_hbm.at[sb[b].at[0]], xb[b], sx[b])
            pltpu.async_copy(ea_hbm.at[pl.ds((start + i) * _CHUNK, _CHUNK)],
                             eb[b], se[b])

        def consume(i, b):
            pltpu.make_async_copy(x_hbm.at[sb[b].at[0]], xb[b], sx[b]).wait()
            pltpu.make_async_copy(
                ea_hbm.at[pl.ds((start + i) * _CHUNK, _CHUNK)],
                eb[b], se[b]).wait()

            def mulrow(j, _):
                for k in range(_D // _LANES):
                    sl = pl.ds(k * _LANES, _LANES)
                    eb[b][j, sl] = eb[b][j, sl] * xb[b][j, sl]
                return 0

            lax.fori_loop(0, _CHUNK, mulrow, 0)
            pltpu.sync_copy(eb[b], acc.at[db[b].at[0]], add=True)

        issue_idx(0, 0)
        issue_idx(1, 1)
        wait_idx(0, 0)
        issue_data(0, 0)

        def pairbody(j, _):
            i0 = 2 * j
            wait_idx(i0 + 1, 1)
            issue_data(i0 + 1, 1)
            consume(i0, 0)
            issue_idx(i0 + 2, 0)
            consume(i0 + 1, 1)
            wait_idx(i0 + 2, 0)
            issue_data(i0 + 2, 0)
            issue_idx(i0 + 3, 1)
            return 0

        npair = _CW // 2 - 1
        lax.fori_loop(0, npair, pairbody, 0)
        i0 = 2 * npair
        wait_idx(i0 + 1, 1)
        issue_data(i0 + 1, 1)
        consume(i0, 0)
        issue_idx(i0 + 2, 0)
        consume(i0 + 1, 1)
        wait_idx(i0 + 2, 0)
        issue_data(i0 + 2, 0)
        consume(i0 + 2, 0)

        plsc.subcore_barrier()
        pltpu.sync_copy(acc.at[pl.ds(rbase, _RPS)],
                        part_hbm.at[c, pl.ds(rbase, _RPS)])

    return pl.kernel(
        body,
        out_type=jax.ShapeDtypeStruct((_NC, _NP, _D), jnp.float32),
        mesh=_mesh(),
        scratch_types=scratch,
    )


@functools.cache
def _make_count_pass():
    scratch = [
        pltpu.VMEM((1, _CHUNK), jnp.int32),
        pltpu.VMEM((1, _CHUNK), jnp.int32),
        pltpu.VMEM((_CHUNK, _D), jnp.float32),
        pltpu.VMEM_SHARED((_NP, _D), jnp.float32),
        pltpu.SemaphoreType.DMA,
        pltpu.SemaphoreType.DMA,
    ]

    def body(dst_hbm, cntp_hbm, db0, db1, oneb, accc, sd0, sd1):
        c = lax.axis_index("c")
        s = lax.axis_index("s")
        db = (db0, db1)
        sd = (sd0, sd1)

        _zero_rows(oneb, _CHUNK, _D)
        rbase = s * _RPS
        for j in range(_RPS // _CHUNK):
            pltpu.sync_copy(oneb, accc.at[pl.ds(rbase + j * _CHUNK, _CHUNK)])
        ov = jnp.ones((_LANES,), jnp.float32)

        def onerow(i, _):
            oneb[i, pl.ds(0, _LANES)] = ov
            return 0

        lax.fori_loop(0, _CHUNK, onerow, 0)
        plsc.subcore_barrier()

        start = c * _CPC + s * _CW

        def issue_idx(i, b):
            pltpu.async_copy(dst_hbm.at[start + i], db[b], sd[b])

        def wait_idx(i, b):
            pltpu.make_async_copy(dst_hbm.at[start + i], db[b], sd[b]).wait()

        def scat(b):
            pltpu.sync_copy(oneb, accc.at[db[b].at[0]], add=True)

        issue_idx(0, 0)

        def pairbody(j, _):
            i0 = 2 * j
            issue_idx(i0 + 1, 1)
            wait_idx(i0, 0)
            scat(0)
            issue_idx(i0 + 2, 0)
            wait_idx(i0 + 1, 1)
            scat(1)
            return 0

        npair = _CW // 2 - 1
        lax.fori_loop(0, npair, pairbody, 0)
        i0 = 2 * npair
        issue_idx(i0 + 1, 1)
        wait_idx(i0, 0)
        scat(0)
        issue_idx(i0 + 2, 0)
        wait_idx(i0 + 1, 1)
        scat(1)
        wait_idx(i0 + 2, 0)
        scat(0)
        plsc.subcore_barrier()

        pltpu.sync_copy(accc.at[pl.ds(rbase, _RPS)],
                        cntp_hbm.at[c, pl.ds(rbase, _RPS)])

    return pl.kernel(
        body,
        out_type=jax.ShapeDtypeStruct((_NC, _NP, _D), jnp.float32),
        mesh=_mesh(),
        scratch_types=scratch,
    )


_BE = 2000


def _rel_chain_body(ea_ref, w0, w1, w2, b0, b1, b2, o1, o2, o3):
    e1 = jnp.dot(ea_ref[...], w0[...],
                 preferred_element_type=jnp.float32) + b0[...]
    e2 = jnp.dot(e1, w1[...], preferred_element_type=jnp.float32) + b1[...]
    e3 = jnp.dot(e2, w2[...], preferred_element_type=jnp.float32) + b2[...]
    o1[...] = e1
    o2[...] = e2
    o3[...] = e3


def _rel_chain(ea, w0t, w1t, w2t, b0, b1, b2):
    ew = pl.BlockSpec((_BE, _D), lambda i: (i, 0))
    wf = pl.BlockSpec((_D, _D), lambda i: (0, 0))
    bf = pl.BlockSpec((1, _D), lambda i: (0, 0))
    return pl.pallas_call(
        _rel_chain_body,
        grid=(_E // _BE,),
        in_specs=[ew, wf, wf, wf, bf, bf, bf],
        out_specs=[ew, ew, ew],
        out_shape=[jax.ShapeDtypeStruct((_E, _D), jnp.float32)] * 3,
    )(ea, w0t, w1t, w2t, b0, b1, b2)


def _combine_body(apply_bn, part, cntp, x, wint, wselft, b_in, b_self,
                  *rest):
    if apply_bn:
        bn_g, bn_b, out = rest
    else:
        (out,) = rest
    seg = part[0, :_N] + part[1, :_N]
    cnt = cntp[0, :_N, 0:1] + cntp[1, :_N, 0:1]
    denom = jnp.maximum(cnt, 1.0)
    agg = (jnp.dot(seg, wint[...], preferred_element_type=jnp.float32) / denom
           + jnp.where(cnt > 0, 1.0, 0.0) * b_in[...])
    val = agg + jnp.dot(x[...], wselft[...],
                        preferred_element_type=jnp.float32) + b_self[...]
    if apply_bn:
        mu = jnp.mean(val, axis=0, keepdims=True)
        var = jnp.mean((val - mu) ** 2, axis=0, keepdims=True)
        val = (val - mu) * lax.rsqrt(var + _EPS) * bn_g[...] + bn_b[...]
        val = jnp.maximum(val, 0.0)
    out[...] = val


def _combine(apply_bn, part, cntp, x, wint, wselft, b_in, b_self, *bn):
    return pl.pallas_call(
        functools.partial(_combine_body, apply_bn),
        out_shape=jax.ShapeDtypeStruct((_N, _D), jnp.float32),
    )(part, cntp, x, wint, wselft, b_in, b_self, *bn)


def kernel(x, edge_index, edge_attr, params):
    src = edge_index[0].reshape(_NCHUNK, 1, _CHUNK)
    dst = edge_index[1].reshape(_NCHUNK, 1, _CHUNK)
    p = params
    row = lambda v: v.reshape(1, _D)

    ea1, ea2, ea3 = _rel_chain(
        edge_attr,
        p["W_rel_0"].T, p["W_rel_1"].T, p["W_rel_2"].T,
        row(p["b_rel_0"]), row(p["b_rel_1"]), row(p["b_rel_2"]),
    )

    cntp = _make_count_pass()(dst)
    part0 = _make_edge_pass()(src, dst, x, edge_attr)
    x1 = _combine(True, part0, cntp, x,
                  p["W_in_0"].T, p["W_self_0"].T,
                  row(p["b_in_0"]), row(p["b_self_0"]),
                  row(p["bn_g_0"]), row(p["bn_b_0"]))

    part1 = _make_edge_pass()(src, dst, x1, ea1)
    x2 = _combine(True, part1, cntp, x1,
                  p["W_in_1"].T, p["W_self_1"].T,
                  row(p["b_in_1"]), row(p["b_self_1"]),
                  row(p["bn_g_1"]), row(p["bn_b_1"]))

    part2 = _make_edge_pass()(src, dst, x2, ea2)
    x3 = _combine(False, part2, cntp, x2,
                  p["W_in_2"].T, p["W_self_2"].T,
                  row(p["b_in_2"]), row(p["b_self_2"]))

    return (x3, ea3)


# speedup vs baseline: 5.5796x; 1.0087x over previous
import functools

import jax
import jax.numpy as jnp
from jax import lax
from jax.experimental import pallas as pl
from jax.experimental.pallas import tpu as pltpu
from jax.experimental.pallas import tpu_sc as plsc

_N = 10000
_E = 320000
_D = 128
_EPS = 1e-5

_NC = 2
_NS = 16
_LANES = 16
_CHUNK = 80
_NCHUNK = _E // _CHUNK
_CPC = _NCHUNK // _NC
_CW = _CPC // _NS
_NP = 10240
_RPS = _NP // _NS


def _zero_rows(buf, rows, cols):
    zv = jnp.zeros((_LANES,), jnp.float32)

    def row(i, _):
        for k in range(cols // _LANES):
            buf[i, pl.ds(k * _LANES, _LANES)] = zv
        return 0

    lax.fori_loop(0, rows, row, 0)


def _mesh():
    return plsc.VectorSubcoreMesh(core_axis_name="c", subcore_axis_name="s",
                                  num_cores=_NC, num_subcores=_NS)


@functools.cache
def _make_edge_pass():
    scratch = [
        pltpu.VMEM((1, _CHUNK), jnp.int32),
        pltpu.VMEM((1, _CHUNK), jnp.int32),
        pltpu.VMEM((1, _CHUNK), jnp.int32),
        pltpu.VMEM((1, _CHUNK), jnp.int32),
        pltpu.VMEM((_CHUNK, _D), jnp.float32),
        pltpu.VMEM((_CHUNK, _D), jnp.float32),
        pltpu.VMEM((_CHUNK, _D), jnp.float32),
        pltpu.VMEM((_CHUNK, _D), jnp.float32),
        pltpu.VMEM_SHARED((_NP, _D), jnp.float32),
        pltpu.SemaphoreType.DMA,
        pltpu.SemaphoreType.DMA,
        pltpu.SemaphoreType.DMA,
        pltpu.SemaphoreType.DMA,
        pltpu.SemaphoreType.DMA,
        pltpu.SemaphoreType.DMA,
        pltpu.SemaphoreType.DMA,
        pltpu.SemaphoreType.DMA,
        pltpu.SemaphoreType.DMA,
        pltpu.SemaphoreType.DMA,
    ]

    def body(src_hbm, dst_hbm, x_hbm, ea_hbm, part_hbm,
             sb0, sb1, db0, db1, xb0, xb1, eb0, eb1, acc,
             ss0, ss1, sd0, sd1, sx0, sx1, se0, se1, sc0, sc1):
        c = lax.axis_index("c")
        s = lax.axis_index("s")
        sb = (sb0, sb1)
        db = (db0, db1)
        xb = (xb0, xb1)
        eb = (eb0, eb1)
        ss = (ss0, ss1)
        sd = (sd0, sd1)
        sx = (sx0, sx1)
        se = (se0, se1)
        sc = (sc0, sc1)

        _zero_rows(xb0, _CHUNK, _D)
        rbase = s * _RPS
        for j in range(_RPS // _CHUNK):
            pltpu.sync_copy(xb0, acc.at[pl.ds(rbase + j * _CHUNK, _CHUNK)])
        plsc.subcore_barrier()

        start = c * _CPC + s * _CW

        def issue_idx(i, b):
            pltpu.async_copy(src_hbm.at[start + i], sb[b], ss[b])
            pltpu.async_copy(dst_hbm.at[start + i], db[b], sd[b])

        def wait_idx(i, b):
            pltpu.make_async_copy(src_hbm.at[start + i], sb[b], ss[b]).wait()
            pltpu.make_async_copy(dst_hbm.at[start + i], db[b], sd[b]).wait()

        def issue_data(i, b):
            pltpu.async_copy(x_hbm.at[sb[b].at[0]], xb[b], sx[b])
            pltpu.async_copy(ea_hbm.at[pl.ds((start + i) * _CHUNK, _CHUNK)],
                             eb[b], se[b])

        def wait_data(i, b):
            pltpu.make_async_copy(x_hbm.at[sb[b].at[0]], xb[b], sx[b]).wait()
            pltpu.make_async_copy(
                ea_hbm.at[pl.ds((start + i) * _CHUNK, _CHUNK)],
                eb[b], se[b]).wait()

        def mult(b):
            def mulrow(j, _):
                for r in range(4):
                    for k in range(_D // _LANES):
                        sl = pl.ds(k * _LANES, _LANES)
                        eb[b][4 * j + r, sl] = (eb[b][4 * j + r, sl]
                                                * xb[b][4 * j + r, sl])
                return 0

            lax.fori_loop(0, _CHUNK // 4, mulrow, 0)

        def scat(b):
            return pltpu.async_copy(eb[b], acc.at[db[b].at[0]], sc[b],
                                    add=True)

        issue_idx(0, 0)
        issue_idx(1, 1)
        wait_idx(0, 0)
        issue_data(0, 0)

        def pairbody(j, _):
            i0 = 2 * j
            wait_idx(i0 + 1, 1)
            issue_data(i0 + 1, 1)
            wait_data(i0, 0)
            mult(0)
            d0 = scat(0)
            wait_data(i0 + 1, 1)
            mult(1)
            d1 = scat(1)
            d0.wait()
            issue_idx(i0 + 2, 0)
            wait_idx(i0 + 2, 0)
            issue_data(i0 + 2, 0)
            d1.wait()
            issue_idx(i0 + 3, 1)
            return 0

        npair = _CW // 2 - 1
        lax.fori_loop(0, npair, pairbody, 0)
        i0 = 2 * npair
        wait_idx(i0 + 1, 1)
        issue_data(i0 + 1, 1)
        wait_data(i0, 0)
        mult(0)
        d0 = scat(0)
        wait_data(i0 + 1, 1)
        mult(1)
        d1 = scat(1)
        d0.wait()
        issue_idx(i0 + 2, 0)
        wait_idx(i0 + 2, 0)
        issue_data(i0 + 2, 0)
        d1.wait()
        wait_data(i0 + 2, 0)
        mult(0)
        scat(0).wait()

        plsc.subcore_barrier()
        pltpu.sync_copy(acc.at[pl.ds(rbase, _RPS)],
                        part_hbm.at[c, pl.ds(rbase, _RPS)])

    return pl.kernel(
        body,
        out_type=jax.ShapeDtypeStruct((_NC, _NP, _D), jnp.float32),
        mesh=_mesh(),
        scratch_types=scratch,
    )


@functools.cache
def _make_count_pass():
    scratch = [
        pltpu.VMEM((1, _CHUNK), jnp.int32),
        pltpu.VMEM((1, _CHUNK), jnp.int32),
        pltpu.VMEM((_CHUNK, _D), jnp.float32),
        pltpu.VMEM_SHARED((_NP, _D), jnp.float32),
        pltpu.SemaphoreType.DMA,
        pltpu.SemaphoreType.DMA,
    ]

    def body(dst_hbm, cntp_hbm, db0, db1, oneb, accc, sd0, sd1):
        c = lax.axis_index("c")
        s = lax.axis_index("s")
        db = (db0, db1)
        sd = (sd0, sd1)

        _zero_rows(oneb, _CHUNK, _D)
        rbase = s * _RPS
        for j in range(_RPS // _CHUNK):
            pltpu.sync_copy(oneb, accc.at[pl.ds(rbase + j * _CHUNK, _CHUNK)])
        ov = jnp.ones((_LANES,), jnp.float32)

        def onerow(i, _):
            oneb[i, pl.ds(0, _LANES)] = ov
            return 0

        lax.fori_loop(0, _CHUNK, onerow, 0)
        plsc.subcore_barrier()

        start = c * _CPC + s * _CW

        def issue_idx(i, b):
            pltpu.async_copy(dst_hbm.at[start + i], db[b], sd[b])

        def wait_idx(i, b):
            pltpu.make_async_copy(dst_hbm.at[start + i], db[b], sd[b]).wait()

        def scat(b):
            pltpu.sync_copy(oneb, accc.at[db[b].at[0]], add=True)

        issue_idx(0, 0)

        def pairbody(j, _):
            i0 = 2 * j
            issue_idx(i0 + 1, 1)
            wait_idx(i0, 0)
            scat(0)
            issue_idx(i0 + 2, 0)
            wait_idx(i0 + 1, 1)
            scat(1)
            return 0

        npair = _CW // 2 - 1
        lax.fori_loop(0, npair, pairbody, 0)
        i0 = 2 * npair
        issue_idx(i0 + 1, 1)
        wait_idx(i0, 0)
        scat(0)
        issue_idx(i0 + 2, 0)
        wait_idx(i0 + 1, 1)
        scat(1)
        wait_idx(i0 + 2, 0)
        scat(0)
        plsc.subcore_barrier()

        pltpu.sync_copy(accc.at[pl.ds(rbase, _RPS)],
                        cntp_hbm.at[c, pl.ds(rbase, _RPS)])

    return pl.kernel(
        body,
        out_type=jax.ShapeDtypeStruct((_NC, _NP, _D), jnp.float32),
        mesh=_mesh(),
        scratch_types=scratch,
    )


_BE = 2000


def _rel_chain_body(ea_ref, w0, w1, w2, b0, b1, b2, o1, o2, o3):
    e1 = jnp.dot(ea_ref[...], w0[...],
                 preferred_element_type=jnp.float32) + b0[...]
    e2 = jnp.dot(e1, w1[...], preferred_element_type=jnp.float32) + b1[...]
    e3 = jnp.dot(e2, w2[...], preferred_element_type=jnp.float32) + b2[...]
    o1[...] = e1
    o2[...] = e2
    o3[...] = e3


def _rel_chain(ea, w0t, w1t, w2t, b0, b1, b2):
    ew = pl.BlockSpec((_BE, _D), lambda i: (i, 0))
    wf = pl.BlockSpec((_D, _D), lambda i: (0, 0))
    bf = pl.BlockSpec((1, _D), lambda i: (0, 0))
    return pl.pallas_call(
        _rel_chain_body,
        grid=(_E // _BE,),
        in_specs=[ew, wf, wf, wf, bf, bf, bf],
        out_specs=[ew, ew, ew],
        out_shape=[jax.ShapeDtypeStruct((_E, _D), jnp.float32)] * 3,
    )(ea, w0t, w1t, w2t, b0, b1, b2)


def _combine_body(apply_bn, part, cntp, x, wint, wselft, b_in, b_self,
                  *rest):
    if apply_bn:
        bn_g, bn_b, out = rest
    else:
        (out,) = rest
    seg = part[0, :_N] + part[1, :_N]
    cnt = cntp[0, :_N, 0:1] + cntp[1, :_N, 0:1]
    denom = jnp.maximum(cnt, 1.0)
    agg = (jnp.dot(seg, wint[...], preferred_element_type=jnp.float32) / denom
           + jnp.where(cnt > 0, 1.0, 0.0) * b_in[...])
    val = agg + jnp.dot(x[...], wselft[...],
                        preferred_element_type=jnp.float32) + b_self[...]
    if apply_bn:
        mu = jnp.mean(val, axis=0, keepdims=True)
        var = jnp.mean((val - mu) ** 2, axis=0, keepdims=True)
        val = (val - mu) * lax.rsqrt(var + _EPS) * bn_g[...] + bn_b[...]
        val = jnp.maximum(val, 0.0)
    out[...] = val


def _combine(apply_bn, part, cntp, x, wint, wselft, b_in, b_self, *bn):
    return pl.pallas_call(
        functools.partial(_combine_body, apply_bn),
        out_shape=jax.ShapeDtypeStruct((_N, _D), jnp.float32),
    )(part, cntp, x, wint, wselft, b_in, b_self, *bn)


def kernel(x, edge_index, edge_attr, params):
    src = edge_index[0].reshape(_NCHUNK, 1, _CHUNK)
    dst = edge_index[1].reshape(_NCHUNK, 1, _CHUNK)
    p = params
    row = lambda v: v.reshape(1, _D)

    ea1, ea2, ea3 = _rel_chain(
        edge_attr,
        p["W_rel_0"].T, p["W_rel_1"].T, p["W_rel_2"].T,
        row(p["b_rel_0"]), row(p["b_rel_1"]), row(p["b_rel_2"]),
    )

    cntp = _make_count_pass()(dst)
    part0 = _make_edge_pass()(src, dst, x, edge_attr)
    x1 = _combine(True, part0, cntp, x,
                  p["W_in_0"].T, p["W_self_0"].T,
                  row(p["b_in_0"]), row(p["b_self_0"]),
                  row(p["bn_g_0"]), row(p["bn_b_0"]))

    part1 = _make_edge_pass()(src, dst, x1, ea1)
    x2 = _combine(True, part1, cntp, x1,
                  p["W_in_1"].T, p["W_self_1"].T,
                  row(p["b_in_1"]), row(p["b_self_1"]),
                  row(p["bn_g_1"]), row(p["bn_b_1"]))

    part2 = _make_edge_pass()(src, dst, x2, ea2)
    x3 = _combine(False, part2, cntp, x2,
                  p["W_in_2"].T, p["W_self_2"].T,
                  row(p["b_in_2"]), row(p["b_self_2"]))

    return (x3, ea3)


# speedup vs baseline: 5.8043x; 1.0403x over previous
import functools

import jax
import jax.numpy as jnp
from jax import lax
from jax.experimental import pallas as pl
from jax.experimental.pallas import tpu as pltpu
from jax.experimental.pallas import tpu_sc as plsc

_N = 10000
_E = 320000
_D = 128
_EPS = 1e-5

_NC = 2
_NS = 16
_LANES = 16
_CHUNK = 80
_NCHUNK = _E // _CHUNK
_CPC = _NCHUNK // _NC
_CW = _CPC // _NS
_NP = 10240
_RPS = _NP // _NS


def _zero_rows(buf, rows, cols):
    zv = jnp.zeros((_LANES,), jnp.float32)

    def row(i, _):
        for k in range(cols // _LANES):
            buf[i, pl.ds(k * _LANES, _LANES)] = zv
        return 0

    lax.fori_loop(0, rows, row, 0)


def _mesh():
    return plsc.VectorSubcoreMesh(core_axis_name="c", subcore_axis_name="s",
                                  num_cores=_NC, num_subcores=_NS)


_IB = 25


@functools.cache
def _make_edge_pass():
    scratch = [
        pltpu.VMEM((_IB, 1, _CHUNK), jnp.int32),
        pltpu.VMEM((_IB, 1, _CHUNK), jnp.int32),
        pltpu.VMEM((_CHUNK, _D), jnp.float32),
        pltpu.VMEM((_CHUNK, _D), jnp.float32),
        pltpu.VMEM((_CHUNK, _D), jnp.float32),
        pltpu.VMEM((_CHUNK, _D), jnp.float32),
        pltpu.VMEM_SHARED((_NP, _D), jnp.float32),
        pltpu.SemaphoreType.DMA,
        pltpu.SemaphoreType.DMA,
        pltpu.SemaphoreType.DMA,
        pltpu.SemaphoreType.DMA,
        pltpu.SemaphoreType.DMA,
        pltpu.SemaphoreType.DMA,
    ]

    def body(src_hbm, dst_hbm, x_hbm, ea_hbm, part_hbm,
             sbk, dbk, xb0, xb1, eb0, eb1, acc,
             sx0, sx1, se0, se1, sc0, sc1):
        c = lax.axis_index("c")
        s = lax.axis_index("s")
        xb = (xb0, xb1)
        eb = (eb0, eb1)
        sx = (sx0, sx1)
        se = (se0, se1)
        sc = (sc0, sc1)

        _zero_rows(xb0, _CHUNK, _D)
        rbase = s * _RPS
        for j in range(_RPS // _CHUNK):
            pltpu.sync_copy(xb0, acc.at[pl.ds(rbase + j * _CHUNK, _CHUNK)])
        plsc.subcore_barrier()

        start = c * _CPC + s * _CW

        def load_idx_block(blk):
            q = start + blk * _IB
            pltpu.sync_copy(src_hbm.at[pl.ds(q, _IB)], sbk)
            pltpu.sync_copy(dst_hbm.at[pl.ds(q, _IB)], dbk)

        def issue_data(blk, j, b):
            pltpu.async_copy(x_hbm.at[sbk.at[j, 0]], xb[b], sx[b])
            q = start + blk * _IB + j
            pltpu.async_copy(ea_hbm.at[pl.ds(q * _CHUNK, _CHUNK)],
                             eb[b], se[b])

        def wait_data(blk, j, b):
            pltpu.make_async_copy(x_hbm.at[sbk.at[j, 0]], xb[b], sx[b]).wait()
            q = start + blk * _IB + j
            pltpu.make_async_copy(ea_hbm.at[pl.ds(q * _CHUNK, _CHUNK)],
                                  eb[b], se[b]).wait()

        def mult(b):
            def mulrow(j, _):
                for r in range(4):
                    for k in range(_D // _LANES):
                        sl = pl.ds(k * _LANES, _LANES)
                        eb[b][4 * j + r, sl] = (eb[b][4 * j + r, sl]
                                                * xb[b][4 * j + r, sl])
                return 0

            lax.fori_loop(0, _CHUNK // 4, mulrow, 0)

        def scat(j, b):
            return pltpu.async_copy(eb[b], acc.at[dbk.at[j, 0]], sc[b],
                                    add=True)

        for blk in range(_CW // _IB):
            load_idx_block(blk)
            issue_data(blk, 0, 0)

            def pairbody(p, _, blk=blk):
                j0 = 2 * p
                issue_data(blk, j0 + 1, 1)
                wait_data(blk, j0, 0)
                mult(0)
                d0 = scat(j0, 0)
                wait_data(blk, j0 + 1, 1)
                mult(1)
                d1 = scat(j0 + 1, 1)
                d0.wait()
                issue_data(blk, j0 + 2, 0)
                d1.wait()
                return 0

            lax.fori_loop(0, _IB // 2, pairbody, 0)
            wait_data(blk, _IB - 1, 0)
            mult(0)
            scat(_IB - 1, 0).wait()

        plsc.subcore_barrier()
        pltpu.sync_copy(acc.at[pl.ds(rbase, _RPS)],
                        part_hbm.at[c, pl.ds(rbase, _RPS)])

    return pl.kernel(
        body,
        out_type=jax.ShapeDtypeStruct((_NC, _NP, _D), jnp.float32),
        mesh=_mesh(),
        scratch_types=scratch,
    )


@functools.cache
def _make_count_pass():
    scratch = [
        pltpu.VMEM((1, _CHUNK), jnp.int32),
        pltpu.VMEM((1, _CHUNK), jnp.int32),
        pltpu.VMEM((_CHUNK, _D), jnp.float32),
        pltpu.VMEM_SHARED((_NP, _D), jnp.float32),
        pltpu.SemaphoreType.DMA,
        pltpu.SemaphoreType.DMA,
    ]

    def body(dst_hbm, cntp_hbm, db0, db1, oneb, accc, sd0, sd1):
        c = lax.axis_index("c")
        s = lax.axis_index("s")
        db = (db0, db1)
        sd = (sd0, sd1)

        _zero_rows(oneb, _CHUNK, _D)
        rbase = s * _RPS
        for j in range(_RPS // _CHUNK):
            pltpu.sync_copy(oneb, accc.at[pl.ds(rbase + j * _CHUNK, _CHUNK)])
        ov = jnp.ones((_LANES,), jnp.float32)

        def onerow(i, _):
            oneb[i, pl.ds(0, _LANES)] = ov
            return 0

        lax.fori_loop(0, _CHUNK, onerow, 0)
        plsc.subcore_barrier()

        start = c * _CPC + s * _CW

        def issue_idx(i, b):
            pltpu.async_copy(dst_hbm.at[start + i], db[b], sd[b])

        def wait_idx(i, b):
            pltpu.make_async_copy(dst_hbm.at[start + i], db[b], sd[b]).wait()

        def scat(b):
            pltpu.sync_copy(oneb, accc.at[db[b].at[0]], add=True)

        issue_idx(0, 0)

        def pairbody(j, _):
            i0 = 2 * j
            issue_idx(i0 + 1, 1)
            wait_idx(i0, 0)
            scat(0)
            issue_idx(i0 + 2, 0)
            wait_idx(i0 + 1, 1)
            scat(1)
            return 0

        npair = _CW // 2 - 1
        lax.fori_loop(0, npair, pairbody, 0)
        i0 = 2 * npair
        issue_idx(i0 + 1, 1)
        wait_idx(i0, 0)
        scat(0)
        issue_idx(i0 + 2, 0)
        wait_idx(i0 + 1, 1)
        scat(1)
        wait_idx(i0 + 2, 0)
        scat(0)
        plsc.subcore_barrier()

        pltpu.sync_copy(accc.at[pl.ds(rbase, _RPS)],
                        cntp_hbm.at[c, pl.ds(rbase, _RPS)])

    return pl.kernel(
        body,
        out_type=jax.ShapeDtypeStruct((_NC, _NP, _D), jnp.float32),
        mesh=_mesh(),
        scratch_types=scratch,
    )


_BE = 2000


def _rel_chain_body(ea_ref, w0, w1, w2, b0, b1, b2, o1, o2, o3):
    e1 = jnp.dot(ea_ref[...], w0[...],
                 preferred_element_type=jnp.float32) + b0[...]
    e2 = jnp.dot(e1, w1[...], preferred_element_type=jnp.float32) + b1[...]
    e3 = jnp.dot(e2, w2[...], preferred_element_type=jnp.float32) + b2[...]
    o1[...] = e1
    o2[...] = e2
    o3[...] = e3


def _rel_chain(ea, w0t, w1t, w2t, b0, b1, b2):
    ew = pl.BlockSpec((_BE, _D), lambda i: (i, 0))
    wf = pl.BlockSpec((_D, _D), lambda i: (0, 0))
    bf = pl.BlockSpec((1, _D), lambda i: (0, 0))
    return pl.pallas_call(
        _rel_chain_body,
        grid=(_E // _BE,),
        in_specs=[ew, wf, wf, wf, bf, bf, bf],
        out_specs=[ew, ew, ew],
        out_shape=[jax.ShapeDtypeStruct((_E, _D), jnp.float32)] * 3,
    )(ea, w0t, w1t, w2t, b0, b1, b2)


def _combine_body(apply_bn, part, cntp, x, wint, wselft, b_in, b_self,
                  *rest):
    if apply_bn:
        bn_g, bn_b, out = rest
    else:
        (out,) = rest
    seg = part[0, :_N] + part[1, :_N]
    cnt = cntp[0, :_N, 0:1] + cntp[1, :_N, 0:1]
    denom = jnp.maximum(cnt, 1.0)
    agg = (jnp.dot(seg, wint[...], preferred_element_type=jnp.float32) / denom
           + jnp.where(cnt > 0, 1.0, 0.0) * b_in[...])
    val = agg + jnp.dot(x[...], wselft[...],
                        preferred_element_type=jnp.float32) + b_self[...]
    if apply_bn:
        mu = jnp.mean(val, axis=0, keepdims=True)
        var = jnp.mean((val - mu) ** 2, axis=0, keepdims=True)
        val = (val - mu) * lax.rsqrt(var + _EPS) * bn_g[...] + bn_b[...]
        val = jnp.maximum(val, 0.0)
    out[...] = val


def _combine(apply_bn, part, cntp, x, wint, wselft, b_in, b_self, *bn):
    return pl.pallas_call(
        functools.partial(_combine_body, apply_bn),
        out_shape=jax.ShapeDtypeStruct((_N, _D), jnp.float32),
    )(part, cntp, x, wint, wselft, b_in, b_self, *bn)


def kernel(x, edge_index, edge_attr, params):
    src = edge_index[0].reshape(_NCHUNK, 1, _CHUNK)
    dst = edge_index[1].reshape(_NCHUNK, 1, _CHUNK)
    p = params
    row = lambda v: v.reshape(1, _D)

    ea1, ea2, ea3 = _rel_chain(
        edge_attr,
        p["W_rel_0"].T, p["W_rel_1"].T, p["W_rel_2"].T,
        row(p["b_rel_0"]), row(p["b_rel_1"]), row(p["b_rel_2"]),
    )

    cntp = _make_count_pass()(dst)
    part0 = _make_edge_pass()(src, dst, x, edge_attr)
    x1 = _combine(True, part0, cntp, x,
                  p["W_in_0"].T, p["W_self_0"].T,
                  row(p["b_in_0"]), row(p["b_self_0"]),
                  row(p["bn_g_0"]), row(p["bn_b_0"]))

    part1 = _make_edge_pass()(src, dst, x1, ea1)
    x2 = _combine(True, part1, cntp, x1,
                  p["W_in_1"].T, p["W_self_1"].T,
                  row(p["b_in_1"]), row(p["b_self_1"]),
                  row(p["bn_g_1"]), row(p["bn_b_1"]))

    part2 = _make_edge_pass()(src, dst, x2, ea2)
    x3 = _combine(False, part2, cntp, x2,
                  p["W_in_2"].T, p["W_self_2"].T,
                  row(p["b_in_2"]), row(p["b_self_2"]))

    return (x3, ea3)


# speedup vs baseline: 5.8045x; 1.0000x over previous
import functools

import jax
import jax.numpy as jnp
import numpy as np
from jax import lax
from jax.experimental import pallas as pl
from jax.experimental.pallas import tpu as pltpu
from jax.experimental.pallas import tpu_sc as plsc

_N = 10000
_E = 320000
_D = 128
_EPS = 1e-5

_NC = 2
_NS = 16
_LANES = 16
_CHUNK = 80
_NCHUNK = _E // _CHUNK
_CPC = _NCHUNK // _NC
_CW = _CPC // _NS
_NP = 10240
_RPS = _NP // _NS


def _zero_rows(buf, rows, cols):
    zv = jnp.zeros((_LANES,), jnp.float32)

    def row(i, _):
        for k in range(cols // _LANES):
            buf[i, pl.ds(k * _LANES, _LANES)] = zv
        return 0

    lax.fori_loop(0, rows, row, 0)


def _mesh():
    return plsc.VectorSubcoreMesh(core_axis_name="c", subcore_axis_name="s",
                                  num_cores=_NC, num_subcores=_NS)


_IB = 25


@functools.cache
def _make_edge_pass():
    scratch = [
        pltpu.VMEM((_IB, 1, _CHUNK), jnp.int32),
        pltpu.VMEM((_IB, 1, _CHUNK), jnp.int32),
        pltpu.VMEM((_CHUNK, _D), jnp.float32),
        pltpu.VMEM((_CHUNK, _D), jnp.float32),
        pltpu.VMEM((_CHUNK, _D), jnp.float32),
        pltpu.VMEM((_CHUNK, _D), jnp.float32),
        pltpu.VMEM_SHARED((_NP, _D), jnp.float32),
        pltpu.SemaphoreType.DMA,
        pltpu.SemaphoreType.DMA,
        pltpu.SemaphoreType.DMA,
        pltpu.SemaphoreType.DMA,
        pltpu.SemaphoreType.DMA,
        pltpu.SemaphoreType.DMA,
    ]

    def body(src_hbm, dst_hbm, x_hbm, ea_hbm, part_hbm,
             sbk, dbk, xb0, xb1, eb0, eb1, acc,
             sx0, sx1, se0, se1, sc0, sc1):
        c = lax.axis_index("c")
        s = lax.axis_index("s")
        xb = (xb0, xb1)
        eb = (eb0, eb1)
        sx = (sx0, sx1)
        se = (se0, se1)
        sc = (sc0, sc1)

        _zero_rows(eb0, _CHUNK, _D)
        rbase = s * _RPS
        for j in range(_RPS // _CHUNK):
            pltpu.sync_copy(eb0, acc.at[pl.ds(rbase + j * _CHUNK, _CHUNK)])
        plsc.subcore_barrier()

        start = c * _CPC + s * _CW

        def load_idx_block(blk):
            q = start + blk * _IB
            pltpu.sync_copy(src_hbm.at[pl.ds(q, _IB)], sbk)
            pltpu.sync_copy(dst_hbm.at[pl.ds(q, _IB)], dbk)

        def issue_data(blk, j, b):
            pltpu.async_copy(x_hbm.at[sbk.at[j, 0]], xb[b], sx[b])
            q = start + blk * _IB + j
            pltpu.async_copy(ea_hbm.at[pl.ds(q * _CHUNK, _CHUNK)],
                             eb[b], se[b])

        def wait_data(blk, j, b):
            pltpu.make_async_copy(x_hbm.at[sbk.at[j, 0]], xb[b], sx[b]).wait()
            q = start + blk * _IB + j
            pltpu.make_async_copy(ea_hbm.at[pl.ds(q * _CHUNK, _CHUNK)],
                                  eb[b], se[b]).wait()

        def mult(b):
            def mulrow(j, _):
                for r in range(4):
                    row = 4 * j + r
                    for k in range(_D // _LANES):
                        sl = pl.ds(k * _LANES, _LANES)
                        eb[b][row, sl] = eb[b][row, sl] * xb[b][row, sl]
                return 0

            lax.fori_loop(0, _CHUNK // 4, mulrow, 0)

        def scat(j, b):
            return pltpu.async_copy(eb[b], acc.at[dbk.at[j, 0]], sc[b],
                                    add=True)

        for blk in range(_CW // _IB):
            load_idx_block(blk)
            issue_data(blk, 0, 0)

            def pairbody(p, _, blk=blk):
                j0 = 2 * p
                issue_data(blk, j0 + 1, 1)
                wait_data(blk, j0, 0)
                mult(0)
                d0 = scat(j0, 0)
                wait_data(blk, j0 + 1, 1)
                mult(1)
                d1 = scat(j0 + 1, 1)
                d0.wait()
                issue_data(blk, j0 + 2, 0)
                d1.wait()
                return 0

            lax.fori_loop(0, _IB // 2, pairbody, 0)
            wait_data(blk, _IB - 1, 0)
            mult(0)
            scat(_IB - 1, 0).wait()

        plsc.subcore_barrier()
        pltpu.sync_copy(acc.at[pl.ds(rbase, _RPS)],
                        part_hbm.at[c, pl.ds(rbase, _RPS)])

    return pl.kernel(
        body,
        out_type=jax.ShapeDtypeStruct((_NC, _NP, _D), jnp.float32),
        mesh=_mesh(),
        scratch_types=scratch,
    )


@functools.cache
def _make_count_pass():
    scratch = [
        pltpu.VMEM((1, _CHUNK), jnp.int32),
        pltpu.VMEM((1, _CHUNK), jnp.int32),
        pltpu.VMEM((_CHUNK, _D), jnp.float32),
        pltpu.VMEM_SHARED((_NP, _D), jnp.float32),
        pltpu.SemaphoreType.DMA,
        pltpu.SemaphoreType.DMA,
    ]

    def body(dst_hbm, cntp_hbm, db0, db1, oneb, accc, sd0, sd1):
        c = lax.axis_index("c")
        s = lax.axis_index("s")
        db = (db0, db1)
        sd = (sd0, sd1)

        _zero_rows(oneb, _CHUNK, _D)
        rbase = s * _RPS
        for j in range(_RPS // _CHUNK):
            pltpu.sync_copy(oneb, accc.at[pl.ds(rbase + j * _CHUNK, _CHUNK)])
        ov = jnp.ones((_LANES,), jnp.float32)

        def onerow(i, _):
            oneb[i, pl.ds(0, _LANES)] = ov
            return 0

        lax.fori_loop(0, _CHUNK, onerow, 0)
        plsc.subcore_barrier()

        start = c * _CPC + s * _CW

        def issue_idx(i, b):
            pltpu.async_copy(dst_hbm.at[start + i], db[b], sd[b])

        def wait_idx(i, b):
            pltpu.make_async_copy(dst_hbm.at[start + i], db[b], sd[b]).wait()

        def scat(b):
            pltpu.sync_copy(oneb, accc.at[db[b].at[0]], add=True)

        issue_idx(0, 0)

        def pairbody(j, _):
            i0 = 2 * j
            issue_idx(i0 + 1, 1)
            wait_idx(i0, 0)
            scat(0)
            issue_idx(i0 + 2, 0)
            wait_idx(i0 + 1, 1)
            scat(1)
            return 0

        npair = _CW // 2 - 1
        lax.fori_loop(0, npair, pairbody, 0)
        i0 = 2 * npair
        issue_idx(i0 + 1, 1)
        wait_idx(i0, 0)
        scat(0)
        issue_idx(i0 + 2, 0)
        wait_idx(i0 + 1, 1)
        scat(1)
        wait_idx(i0 + 2, 0)
        scat(0)
        plsc.subcore_barrier()

        pltpu.sync_copy(accc.at[pl.ds(rbase, _RPS)],
                        cntp_hbm.at[c, pl.ds(rbase, _RPS)])

    return pl.kernel(
        body,
        out_type=jax.ShapeDtypeStruct((_NC, _NP, _D), jnp.float32),
        mesh=_mesh(),
        scratch_types=scratch,
    )


_BE = 2000


def _rel_chain_body(ea_ref, w0, w1, w2, b0, b1, b2, o1, o2, o3):
    e1 = jnp.dot(ea_ref[...], w0[...],
                 preferred_element_type=jnp.float32) + b0[...]
    e2 = jnp.dot(e1, w1[...], preferred_element_type=jnp.float32) + b1[...]
    e3 = jnp.dot(e2, w2[...], preferred_element_type=jnp.float32) + b2[...]
    o1[...] = e1
    o2[...] = e2
    o3[...] = e3


def _rel_chain(ea, w0t, w1t, w2t, b0, b1, b2):
    ew = pl.BlockSpec((_BE, _D), lambda i: (i, 0))
    wf = pl.BlockSpec((_D, _D), lambda i: (0, 0))
    bf = pl.BlockSpec((1, _D), lambda i: (0, 0))
    return pl.pallas_call(
        _rel_chain_body,
        grid=(_E // _BE,),
        in_specs=[ew, wf, wf, wf, bf, bf, bf],
        out_specs=[ew, ew, ew],
        out_shape=[jax.ShapeDtypeStruct((_E, _D), jnp.float32)] * 3,
    )(ea, w0t, w1t, w2t, b0, b1, b2)


def _combine_body(apply_bn, part, cntp, x, wint, wselft, b_in, b_self,
                  *rest):
    if apply_bn:
        bn_g, bn_b, out = rest
    else:
        (out,) = rest
    seg = part[0, :_N] + part[1, :_N]
    cnt = cntp[0, :_N, 0:1] + cntp[1, :_N, 0:1]
    denom = jnp.maximum(cnt, 1.0)
    agg = (jnp.dot(seg, wint[...], preferred_element_type=jnp.float32) / denom
           + jnp.where(cnt > 0, 1.0, 0.0) * b_in[...])
    val = agg + jnp.dot(x[...], wselft[...],
                        preferred_element_type=jnp.float32) + b_self[...]
    if apply_bn:
        mu = jnp.mean(val, axis=0, keepdims=True)
        var = jnp.mean((val - mu) ** 2, axis=0, keepdims=True)
        val = (val - mu) * lax.rsqrt(var + _EPS) * bn_g[...] + bn_b[...]
        val = jnp.maximum(val, 0.0)
    out[...] = val


def _combine(apply_bn, part, cntp, x, wint, wselft, b_in, b_self, *bn):
    return pl.pallas_call(
        functools.partial(_combine_body, apply_bn),
        out_shape=jax.ShapeDtypeStruct((_N, _D), jnp.float32),
    )(part, cntp, x, wint, wselft, b_in, b_self, *bn)


def kernel(x, edge_index, edge_attr, params):
    src = edge_index[0].reshape(_NCHUNK, 1, _CHUNK)
    dst = edge_index[1].reshape(_NCHUNK, 1, _CHUNK)
    p = params
    row = lambda v: v.reshape(1, _D)

    ea1, ea2, ea3 = _rel_chain(
        edge_attr,
        p["W_rel_0"].T, p["W_rel_1"].T, p["W_rel_2"].T,
        row(p["b_rel_0"]), row(p["b_rel_1"]), row(p["b_rel_2"]),
    )

    cntp = _make_count_pass()(dst)
    part0 = _make_edge_pass()(src, dst, x, edge_attr)
    x1 = _combine(True, part0, cntp, x,
                  p["W_in_0"].T, p["W_self_0"].T,
                  row(p["b_in_0"]), row(p["b_self_0"]),
                  row(p["bn_g_0"]), row(p["bn_b_0"]))

    part1 = _make_edge_pass()(src, dst, x1, ea1)
    x2 = _combine(True, part1, cntp, x1,
                  p["W_in_1"].T, p["W_self_1"].T,
                  row(p["b_in_1"]), row(p["b_self_1"]),
                  row(p["bn_g_1"]), row(p["bn_b_1"]))

    part2 = _make_edge_pass()(src, dst, x2, ea2)
    x3 = _combine(False, part2, cntp, x2,
                  p["W_in_2"].T, p["W_self_2"].T,
                  row(p["b_in_2"]), row(p["b_self_2"]))

    return (x3, ea3)
